# baseline (speedup 1.0000x reference)
import functools

import jax
import jax.numpy as jnp
from jax import lax
from jax.experimental import pallas as pl
from jax.experimental.pallas import tpu as pltpu
from jax.experimental.pallas import tpu_sc as plsc

_B, _F, _V = 4, 128, 12288
_ROWS = _B * _F
_V4 = 4 * _V
_N_IN = _ROWS * _V
_N_OUT = _ROWS * _V4
_NW = 32
_LANES = 16
_UNROLL = 4
_CPW = 32
_CH_IN = _N_IN // (_NW * _CPW)
_CH_OUT = 4 * _CH_IN

_mesh = plsc.VectorSubcoreMesh(core_axis_name="c", subcore_axis_name="s")


@functools.partial(
    pl.kernel,
    out_type=jax.ShapeDtypeStruct((_N_OUT,), jnp.float32),
    mesh=_mesh,
    compiler_params=pltpu.CompilerParams(
        needs_layout_passes=False, use_tc_tiling_on_sc=False),
    scratch_types=[
        pltpu.VMEM((_CH_IN,), jnp.float32),
        pltpu.VMEM((_CH_IN,), jnp.float32),
        pltpu.VMEM((_CH_OUT,), jnp.float32),
        pltpu.VMEM((_CH_OUT,), jnp.float32),
        pltpu.SemaphoreType.DMA,
        pltpu.SemaphoreType.DMA,
        pltpu.SemaphoreType.DMA,
        pltpu.SemaphoreType.DMA,
    ],
)
def _unpool_sc(x_hbm, out_hbm, in0, in1, out0, out1, si0, si1, so0, so1):
    wid = lax.axis_index("s") * 2 + lax.axis_index("c")
    ch0 = wid * _CPW
    ins = (in0, in1)
    outs = (out0, out1)
    sins = (si0, si1)
    souts = (so0, so1)

    def xsl(m):
        return x_hbm.at[pl.ds((ch0 + m) * _CH_IN, _CH_IN)]

    def osl(m):
        return out_hbm.at[pl.ds((ch0 + m) * _CH_OUT, _CH_OUT)]

    in_h = [None, None]
    out_h = [None, None]
    in_h[0] = pltpu.async_copy(xsl(0), in0, si0)
    in_h[1] = pltpu.async_copy(xsl(1), in1, si1)
    for m in range(_CPW):
        b = m % 2
        in_h[b].wait()
        if m >= 2:
            out_h[b].wait()
        src = ins[b]
        dst = outs[b]

        @plsc.parallel_loop(0, _CH_OUT // _LANES, step=4, unroll=_UNROLL)
        def body(j, src=src, dst=dst):
            iota4 = lax.iota(jnp.int32, _LANES) // 4
            for q in range(4):
                vals = plsc.load_gather(src, [iota4 + (4 * (j + q))])
                dst[pl.ds((j + q) * _LANES, _LANES)] = vals

        out_h[b] = pltpu.async_copy(dst, osl(m), souts[b])
        if m + 2 < _CPW:
            in_h[b] = pltpu.async_copy(xsl(m + 2), ins[b], sins[b])
    out_h[0].wait()
    out_h[1].wait()


def kernel(x, indices_spa, indices_sph):
    x1 = x.reshape(_N_IN)
    out = _unpool_sc(x1)
    return out.reshape(_B, _F, _V4, 1, 1, 1)


# speedup vs baseline: 1.0558x; 1.0558x over previous
import functools

import jax
import jax.numpy as jnp
from jax import lax
from jax.experimental import pallas as pl
from jax.experimental.pallas import tpu as pltpu
from jax.experimental.pallas import tpu_sc as plsc

_B, _F, _V = 4, 128, 12288
_ROWS = _B * _F
_V4 = 4 * _V
_NW = 32
_RPW = _ROWS // _NW
_LANES = 16
_UNROLL = 4

_mesh = plsc.VectorSubcoreMesh(core_axis_name="c", subcore_axis_name="s")


@functools.partial(
    pl.kernel,
    out_type=jax.ShapeDtypeStruct((_ROWS, _V4), jnp.float32),
    mesh=_mesh,
    compiler_params=pltpu.CompilerParams(needs_layout_passes=False, use_tc_tiling_on_sc=False),
    scratch_types=[
        pltpu.VMEM((_V,), jnp.float32),
        pltpu.VMEM((_V,), jnp.float32),
        pltpu.VMEM((_V4,), jnp.float32),
        pltpu.VMEM((_V4,), jnp.float32),
        pltpu.SemaphoreType.DMA,
        pltpu.SemaphoreType.DMA,
        pltpu.SemaphoreType.DMA,
        pltpu.SemaphoreType.DMA,
    ],
)
def _unpool_sc(x_hbm, out_hbm, in0, in1, out0, out1, si0, si1, so0, so1):
    wid = lax.axis_index("s") * 2 + lax.axis_index("c")
    row0 = wid * _RPW
    ins = (in0, in1)
    outs = (out0, out1)
    sins = (si0, si1)
    souts = (so0, so1)

    in_h = [None, None]
    out_h = [None, None]
    in_h[0] = pltpu.async_copy(x_hbm.at[row0 + 0], in0, si0)
    for r in range(_RPW):
        b = r % 2
        in_h[b].wait()
        if r + 1 < _RPW:
            in_h[1 - b] = pltpu.async_copy(
                x_hbm.at[row0 + r + 1], ins[1 - b], sins[1 - b])
        if r >= 2:
            out_h[b].wait()
        src = ins[b]
        dst = outs[b]

        @plsc.parallel_loop(0, _V4 // _LANES, step=4, unroll=_UNROLL)
        def body(j, src=src, dst=dst):
            iota4 = lax.iota(jnp.int32, _LANES) // 4
            for q in range(4):
                vals = plsc.load_gather(src, [iota4 + (4 * (j + q))])
                dst[pl.ds((j + q) * _LANES, _LANES)] = vals
        out_h[b] = pltpu.async_copy(dst, out_hbm.at[row0 + r], souts[b])
    out_h[0].wait()
    out_h[1].wait()


def kernel(x, indices_spa, indices_sph):
    x2 = x.reshape(_ROWS, _V)
    out = _unpool_sc(x2)
    return out.reshape(_B, _F, _V4, 1, 1, 1)
